# baseline (speedup 1.0000x reference)
import functools

import jax
import jax.numpy as jnp
from jax import lax
from jax.experimental import pallas as pl
from jax.experimental.pallas import tpu as pltpu
from jax.experimental.pallas import tpu_sc as plsc

EMBEDDING_DIM = 32
NUM_CORES = 2
NUM_SUBCORES = 16
NUM_WORKERS = NUM_CORES * NUM_SUBCORES
TILE_B = 128
GROUP = 4


def _make_gather(batch: int, fields: int):
  n_units = fields * (batch // TILE_B)
  per_w = n_units // NUM_WORKERS
  n_groups = per_w // GROUP
  assert n_units % NUM_WORKERS == 0 and per_w % GROUP == 0
  cblk = EMBEDDING_DIM // 8

  mesh = plsc.VectorSubcoreMesh(
      core_axis_name="c", subcore_axis_name="s",
      num_cores=NUM_CORES, num_subcores=NUM_SUBCORES)

  @functools.partial(
      pl.kernel,
      mesh=mesh,
      compiler_params=pltpu.CompilerParams(
          use_tc_tiling_on_sc=False, needs_layout_passes=False),
      out_type=jax.ShapeDtypeStruct(
          (fields, cblk, batch // TILE_B, 8, TILE_B), jnp.float32),
      scratch_types=[
          pltpu.VMEM((GROUP, TILE_B), jnp.int32),
          pltpu.VMEM((GROUP, TILE_B, EMBEDDING_DIM), jnp.float32),
          pltpu.VMEM((GROUP, cblk, 8, TILE_B), jnp.float32),
          pltpu.SemaphoreType.DMA,
          pltpu.SemaphoreType.DMA,
      ],
  )
  def gather_kernel(idx_hbm, tab_hbm, out_hbm, idx_v, rows_v, rowsT_v,
                    sem_g, sem_o):
    wid = lax.axis_index("s") * NUM_CORES + lax.axis_index("c")
    base_u = wid * per_w
    lanes = lax.iota(jnp.int32, 16)

    def do_group(grp, carry):
      u0 = base_u + grp * GROUP
      gathers = []
      for b in range(GROUP):
        off = (u0 + b) * TILE_B
        pltpu.sync_copy(idx_hbm.at[pl.ds(off, TILE_B)], idx_v.at[b])
        gathers.append(
            pltpu.async_copy(tab_hbm.at[idx_v.at[b]], rows_v.at[b], sem_g))
      stores = []
      for b in range(GROUP):
        gathers[b].wait()
        for c in range(EMBEDDING_DIM):
          col = jnp.full((16,), c, jnp.int32)
          for sub in range(TILE_B // 16):
            v = plsc.load_gather(rows_v.at[b], [lanes + sub * 16, col])
            rowsT_v[b, c // 8, c % 8, pl.ds(sub * 16, 16)] = v
        ug = u0 + b
        f = ug // (batch // TILE_B)
        bt = ug - f * (batch // TILE_B)
        stores.append(
            pltpu.async_copy(rowsT_v.at[b], out_hbm.at[f, :, bt], sem_o))
      for s in stores:
        s.wait()
      return carry

    lax.fori_loop(0, n_groups, do_group, 0, unroll=False)

  return gather_kernel


def kernel(input, table):
  batch, fields = input.shape
  idx = input.T.reshape(batch * fields).astype(jnp.int32)
  out = _make_gather(batch, fields)(idx, table)
  return (out.transpose(2, 4, 0, 1, 3)
             .reshape(batch, fields, EMBEDDING_DIM))


# speedup vs baseline: 1.1028x; 1.1028x over previous
import functools

import jax
import jax.numpy as jnp
from jax import lax
from jax.experimental import pallas as pl
from jax.experimental.pallas import tpu as pltpu
from jax.experimental.pallas import tpu_sc as plsc

EMBEDDING_DIM = 32
NUM_CORES = 2
NUM_SUBCORES = 16
NUM_WORKERS = NUM_CORES * NUM_SUBCORES
TILE_B = 128
GROUP = 8


def _make_gather(batch: int, fields: int):
  n_units = fields * (batch // TILE_B)
  per_w = n_units // NUM_WORKERS
  n_groups = per_w // GROUP
  assert n_units % NUM_WORKERS == 0 and per_w % GROUP == 0
  cblk = EMBEDDING_DIM // 8

  mesh = plsc.VectorSubcoreMesh(
      core_axis_name="c", subcore_axis_name="s",
      num_cores=NUM_CORES, num_subcores=NUM_SUBCORES)

  @functools.partial(
      pl.kernel,
      mesh=mesh,
      compiler_params=pltpu.CompilerParams(
          use_tc_tiling_on_sc=False, needs_layout_passes=False),
      out_type=jax.ShapeDtypeStruct(
          (fields, cblk, batch // TILE_B, 8, TILE_B), jnp.float32),
      scratch_types=[
          pltpu.VMEM((GROUP * TILE_B,), jnp.int32),
          pltpu.VMEM((GROUP, TILE_B, EMBEDDING_DIM), jnp.float32),
          pltpu.VMEM((GROUP, cblk, 8, TILE_B), jnp.float32),
          pltpu.SemaphoreType.DMA,
          pltpu.SemaphoreType.DMA,
      ],
  )
  def gather_kernel(idx_hbm, tab_hbm, out_hbm, idx_v, rows_v, rowsT_v,
                    sem_g, sem_o):
    wid = lax.axis_index("s") * NUM_CORES + lax.axis_index("c")
    base_u = wid * per_w
    lanes = lax.iota(jnp.int32, 16)

    cols = [jnp.full((16,), c, jnp.int32) for c in range(EMBEDDING_DIM)]

    def do_group(grp, carry):
      u0 = base_u + grp * GROUP
      pltpu.sync_copy(idx_hbm.at[pl.ds(u0 * TILE_B, GROUP * TILE_B)], idx_v)
      gathers = []
      for b in range(GROUP):
        gathers.append(
            pltpu.async_copy(tab_hbm.at[idx_v.at[pl.ds(b * TILE_B, TILE_B)]],
                             rows_v.at[b], sem_g))
      stores = []
      for b in range(GROUP):
        gathers[b].wait()

        def transpose_sub(sub, _, b=b):
          row_ids = lanes + sub * 16
          for c in range(EMBEDDING_DIM):
            v = plsc.load_gather(rows_v.at[b], [row_ids, cols[c]])
            rowsT_v[b, c // 8, c % 8, pl.ds(sub * 16, 16)] = v
          return _

        lax.fori_loop(0, TILE_B // 16, transpose_sub, 0, unroll=False)
        ug = u0 + b
        f = ug // (batch // TILE_B)
        bt = ug - f * (batch // TILE_B)
        stores.append(
            pltpu.async_copy(rowsT_v.at[b], out_hbm.at[f, :, bt], sem_o))
      for s in stores:
        s.wait()
      return carry

    lax.fori_loop(0, n_groups, do_group, 0, unroll=False)

  return gather_kernel


def kernel(input, table):
  batch, fields = input.shape
  idx = input.T.reshape(batch * fields).astype(jnp.int32)
  out = _make_gather(batch, fields)(idx, table)
  return (out.transpose(2, 4, 0, 1, 3)
             .reshape(batch, fields, EMBEDDING_DIM))


# speedup vs baseline: 1.1807x; 1.0706x over previous
import functools

import jax
import jax.numpy as jnp
from jax import lax
from jax.experimental import pallas as pl
from jax.experimental.pallas import tpu as pltpu
from jax.experimental.pallas import tpu_sc as plsc

EMBEDDING_DIM = 32
NUM_CORES = 2
NUM_SUBCORES = 16
NUM_WORKERS = NUM_CORES * NUM_SUBCORES
CHUNK = 512
NBUF = 7


def _make_gather(num_idx: int):
  per_w = num_idx // NUM_WORKERS
  n_chunks = per_w // CHUNK
  assert per_w % CHUNK == 0 and num_idx % NUM_WORKERS == 0

  mesh = plsc.VectorSubcoreMesh(
      core_axis_name="c", subcore_axis_name="s",
      num_cores=NUM_CORES, num_subcores=NUM_SUBCORES)

  @functools.partial(
      pl.kernel,
      mesh=mesh,
      compiler_params=pltpu.CompilerParams(use_tc_tiling_on_sc=False),
      out_type=jax.ShapeDtypeStruct((num_idx, EMBEDDING_DIM), jnp.float32),
      scratch_types=[
          pltpu.VMEM((NBUF, CHUNK), jnp.int32),
          pltpu.VMEM((NBUF, CHUNK, EMBEDDING_DIM), jnp.float32),
          pltpu.SemaphoreType.DMA,
          pltpu.SemaphoreType.DMA,
      ],
  )
  def gather_kernel(idx_hbm, tab_hbm, out_hbm, idx_v, rows_v, sem_g, sem_o):
    wid = lax.axis_index("s") * NUM_CORES + lax.axis_index("c")
    base = wid * per_w

    gathers = [None] * n_chunks
    stores = [None] * n_chunks

    def store_chunk(g):
      gathers[g].wait()
      stores[g] = pltpu.async_copy(
          rows_v.at[g % NBUF], out_hbm.at[pl.ds(base + g * CHUNK, CHUNK)],
          sem_o)

    for g in range(n_chunks):
      b = g % NBUF
      if g >= NBUF:
        stores[g - NBUF].wait()
      pltpu.sync_copy(idx_hbm.at[pl.ds(base + g * CHUNK, CHUNK)],
                      idx_v.at[b])
      gathers[g] = pltpu.async_copy(tab_hbm.at[idx_v.at[b]], rows_v.at[b],
                                    sem_g)
      if g >= NBUF - 1:
        store_chunk(g - NBUF + 1)
    for g in range(max(0, n_chunks - NBUF + 1), n_chunks):
      store_chunk(g)
    for g in range(max(0, n_chunks - NBUF), n_chunks):
      stores[g].wait()

  return gather_kernel


def kernel(input, table):
  batch, fields = input.shape
  num_idx = batch * fields
  idx = input.reshape(num_idx).astype(jnp.int32)
  out = _make_gather(num_idx)(idx, table)
  return out.reshape(batch, fields, EMBEDDING_DIM)
